# baseline (device time: 74238 ns/iter reference)
import jax
import jax.numpy as jnp
from jax import lax
from jax.experimental import pallas as pl
from jax.experimental.pallas import tpu as pltpu

N_DEV = 4
E_PER = 4
CAPACITY = 204.0


def kernel(x, router_W, route_idx, expert_W):
    del router_W
    m_tok, d_model = x.shape
    _, _, d_ff = expert_W.shape
    n_exp = N_DEV * E_PER

    def body(x_ref, ridx_ref, ew_ref, out_ref,
             cnt_all, myb, bL, bR, bD,
             cnt_send_sems, cnt_recv_sems, w_send_sems, w_recv_sems):
        p = lax.axis_index("i")
        right = lax.rem(p + 1, N_DEV)
        left = lax.rem(p + N_DEV - 1, N_DEV)

        r = ridx_ref[:, :]
        e_iota = lax.broadcasted_iota(jnp.int32, (m_tok, n_exp), 1)
        onehot = (r == e_iota).astype(jnp.float32)
        row = lax.broadcasted_iota(jnp.int32, (m_tok, m_tok), 0)
        col = lax.broadcasted_iota(jnp.int32, (m_tok, m_tok), 1)
        tril = (row > col).astype(jnp.float32)
        excl = jnp.dot(tril, onehot, preferred_element_type=jnp.float32)
        counts_mine = jnp.sum(onehot, axis=0, keepdims=True)
        cnt_all[0, :, 0:n_exp] = counts_mine

        j_iota = lax.broadcasted_iota(jnp.int32, (1, E_PER), 1)
        srow = jnp.zeros((1, E_PER), jnp.float32)
        for j in range(E_PER):
            w = ew_ref[j, :, :]
            s = jnp.max(jnp.abs(w)) / 127.0
            myb[j, :, :] = jnp.round(w / s).astype(jnp.int8)
            srow = srow + jnp.where(j_iota == j, s, 0.0)
        cnt_all[0, :, n_exp:] = srow
        xb = x_ref[:, :].astype(jnp.bfloat16)

        barrier_sem = pltpu.get_barrier_semaphore()
        for k in range(1, N_DEV):
            pl.semaphore_signal(
                barrier_sem, inc=1,
                device_id=(lax.rem(p + k, N_DEV),),
                device_id_type=pl.DeviceIdType.MESH,
            )
        pl.semaphore_wait(barrier_sem, N_DEV - 1)

        H = E_PER // 2
        lo, hi = pl.ds(0, H), pl.ds(H, H)

        def wcopy(src, dst, sem_idx, target):
            return pltpu.make_async_remote_copy(
                src_ref=src, dst_ref=dst,
                send_sem=w_send_sems.at[sem_idx],
                recv_sem=w_recv_sems.at[sem_idx],
                device_id=(target,),
                device_id_type=pl.DeviceIdType.MESH,
            )

        h1r_a = wcopy(myb.at[lo], bL.at[lo], 0, right)
        h1l_a = wcopy(myb.at[hi], bR.at[hi], 2, left)
        h1r_b = wcopy(myb.at[hi], bL.at[hi], 1, right)
        h1l_b = wcopy(myb.at[lo], bR.at[lo], 3, left)
        h1r_a.start()
        h1l_a.start()
        h1r_b.start()
        h1l_b.start()

        cnt_rdmas = []
        for k in range(1, N_DEV):
            rd = pltpu.make_async_remote_copy(
                src_ref=cnt_all.at[0],
                dst_ref=cnt_all.at[k],
                send_sem=cnt_send_sems.at[k - 1],
                recv_sem=cnt_recv_sems.at[k - 1],
                device_id=(lax.rem(p + k, N_DEV),),
                device_id_type=pl.DeviceIdType.MESH,
            )
            rd.start()
            cnt_rdmas.append(rd)
        for rd in cnt_rdmas:
            rd.wait()

        offs = jnp.zeros((1, n_exp), jnp.float32)
        for k in range(1, N_DEV):
            origin = lax.rem(p - k + N_DEV, N_DEV)
            offs = offs + jnp.where(origin < p, cnt_all[k, :, 0:n_exp], 0.0)
        rank_full = excl + offs
        rank_tok = jnp.sum(rank_full * onehot, axis=1, keepdims=True)
        kept = rank_tok < CAPACITY

        def block_out(w_ref, origin, row, js=tuple(range(E_PER))):
            acc = jnp.zeros((m_tok, d_ff), jnp.float32)
            for j in js:
                e = E_PER * origin + j
                s = cnt_all[row, 0, n_exp + j]
                m = jnp.where((r == e) & kept, s, 0.0)
                acc = acc + jnp.dot(
                    xb * m.astype(jnp.bfloat16),
                    w_ref[j, :, :].astype(jnp.bfloat16),
                    preferred_element_type=jnp.float32,
                )
            return acc

        out_ref[:, :] = block_out(myb, p, 0)

        far = lax.rem(p + 2, N_DEV)
        JLO, JHI = tuple(range(H)), tuple(range(H, E_PER))

        h1r_a.wait_recv()
        h2r = wcopy(bL.at[lo], bD.at[lo], 4, right)
        h2r.start()
        out_ref[:, :] = out_ref[:, :] + block_out(bL, left, 1, js=JLO)

        h1l_a.wait_recv()
        h2l = wcopy(bR.at[hi], bD.at[hi], 5, left)
        h2l.start()
        out_ref[:, :] = out_ref[:, :] + block_out(bR, right, 3, js=JHI)

        h1r_b.wait_recv()
        out_ref[:, :] = out_ref[:, :] + block_out(bL, left, 1, js=JHI)
        h1l_b.wait_recv()
        out_ref[:, :] = out_ref[:, :] + block_out(bR, right, 3, js=JLO)

        h2r.wait_recv()
        out_ref[:, :] = out_ref[:, :] + block_out(bD, far, 2, js=JLO)
        h2l.wait_recv()
        out_ref[:, :] = out_ref[:, :] + block_out(bD, far, 2, js=JHI)

        for rd in (h1r_a, h1l_a, h1r_b, h1l_b, h2r, h2l):
            rd.wait_send()

    return pl.pallas_call(
        body,
        out_shape=jax.ShapeDtypeStruct((m_tok, d_ff), jnp.float32),
        in_specs=[
            pl.BlockSpec(memory_space=pltpu.VMEM),
            pl.BlockSpec(memory_space=pltpu.VMEM),
            pl.BlockSpec(memory_space=pltpu.VMEM),
        ],
        out_specs=pl.BlockSpec(memory_space=pltpu.VMEM),
        scratch_shapes=[
            pltpu.VMEM((N_DEV, 1, n_exp + E_PER), jnp.float32),
            pltpu.VMEM((E_PER, d_model, d_ff), jnp.int8),
            pltpu.VMEM((E_PER, d_model, d_ff), jnp.int8),
            pltpu.VMEM((E_PER, d_model, d_ff), jnp.int8),
            pltpu.VMEM((E_PER, d_model, d_ff), jnp.int8),
            pltpu.SemaphoreType.DMA((N_DEV - 1,)),
            pltpu.SemaphoreType.DMA((N_DEV - 1,)),
            pltpu.SemaphoreType.DMA((6,)),
            pltpu.SemaphoreType.DMA((6,)),
        ],
        compiler_params=pltpu.CompilerParams(
            collective_id=0,
            vmem_limit_bytes=64 * 1024 * 1024,
        ),
    )(x, route_idx, expert_W)


# device time: 57059 ns/iter; 1.3011x vs baseline; 1.3011x over previous
import jax
import jax.numpy as jnp
from jax import lax
from jax.experimental import pallas as pl
from jax.experimental.pallas import tpu as pltpu

N_DEV = 4
E_PER = 4
CAPACITY = 204.0


def kernel(x, router_W, route_idx, expert_W):
    del router_W
    m_tok, d_model = x.shape
    _, _, d_ff = expert_W.shape
    n_exp = N_DEV * E_PER

    def body(x_ref, ridx_ref, ew_ref, out_ref,
             cnt_all, myb, bL, bR, bD,
             cnt_send_sems, cnt_recv_sems, w_send_sems, w_recv_sems):
        p = lax.axis_index("i")
        right = lax.rem(p + 1, N_DEV)
        left = lax.rem(p + N_DEV - 1, N_DEV)

        r = ridx_ref[:, :]
        e_iota = lax.broadcasted_iota(jnp.int32, (m_tok, n_exp), 1)
        onehot = (r == e_iota).astype(jnp.float32)
        row = lax.broadcasted_iota(jnp.int32, (m_tok, m_tok), 0)
        col = lax.broadcasted_iota(jnp.int32, (m_tok, m_tok), 1)
        tril = (row > col).astype(jnp.float32)
        excl = jnp.dot(tril, onehot, preferred_element_type=jnp.float32)
        counts_mine = jnp.sum(onehot, axis=0, keepdims=True)
        cnt_all[0, :, 0:n_exp] = counts_mine

        j_iota = lax.broadcasted_iota(jnp.int32, (1, E_PER), 1)
        srow = jnp.zeros((1, E_PER), jnp.float32)
        for j in range(E_PER):
            w = ew_ref[j, :, :]
            s = jnp.max(jnp.abs(w)) / 127.0
            myb[j, :, :] = jnp.round(w / s).astype(jnp.int8)
            srow = srow + jnp.where(j_iota == j, s, 0.0)
        cnt_all[0, :, n_exp:] = srow
        xb = x_ref[:, :].astype(jnp.bfloat16)

        barrier_sem = pltpu.get_barrier_semaphore()
        for k in range(1, N_DEV):
            pl.semaphore_signal(
                barrier_sem, inc=1,
                device_id=(lax.rem(p + k, N_DEV),),
                device_id_type=pl.DeviceIdType.MESH,
            )
        pl.semaphore_wait(barrier_sem, N_DEV - 1)

        cnt_rdmas = []
        for k in range(1, N_DEV):
            rd = pltpu.make_async_remote_copy(
                src_ref=cnt_all.at[0],
                dst_ref=cnt_all.at[k],
                send_sem=cnt_send_sems.at[k - 1],
                recv_sem=cnt_recv_sems.at[k - 1],
                device_id=(lax.rem(p + k, N_DEV),),
                device_id_type=pl.DeviceIdType.MESH,
            )
            rd.start()
            cnt_rdmas.append(rd)

        H = E_PER // 2
        lo, hi = pl.ds(0, H), pl.ds(H, H)

        def wcopy(src, dst, sem_idx, target):
            return pltpu.make_async_remote_copy(
                src_ref=src, dst_ref=dst,
                send_sem=w_send_sems.at[sem_idx],
                recv_sem=w_recv_sems.at[sem_idx],
                device_id=(target,),
                device_id_type=pl.DeviceIdType.MESH,
            )

        h1r_a = wcopy(myb.at[lo], bL.at[lo], 0, right)
        h1l_a = wcopy(myb.at[hi], bR.at[hi], 2, left)
        h1r_b = wcopy(myb.at[hi], bL.at[hi], 1, right)
        h1l_b = wcopy(myb.at[lo], bR.at[lo], 3, left)
        h1r_a.start()
        h1l_a.start()
        h1r_b.start()
        h1l_b.start()

        for rd in cnt_rdmas:
            rd.wait()

        offs = jnp.zeros((1, n_exp), jnp.float32)
        for k in range(1, N_DEV):
            origin = lax.rem(p - k + N_DEV, N_DEV)
            offs = offs + jnp.where(origin < p, cnt_all[k, :, 0:n_exp], 0.0)
        rank_full = excl + offs
        rank_tok = jnp.sum(rank_full * onehot, axis=1, keepdims=True)
        kept = rank_tok < CAPACITY

        def block_out(w_ref, origin, row, js=tuple(range(E_PER))):
            acc = jnp.zeros((m_tok, d_ff), jnp.float32)
            for j in js:
                e = E_PER * origin + j
                s = cnt_all[row, 0, n_exp + j]
                m = jnp.where((r == e) & kept, s, 0.0)
                acc = acc + jnp.dot(
                    xb * m.astype(jnp.bfloat16),
                    w_ref[j, :, :].astype(jnp.bfloat16),
                    preferred_element_type=jnp.float32,
                )
            return acc

        out_ref[:, :] = block_out(myb, p, 0)

        far = lax.rem(p + 2, N_DEV)
        JLO, JHI = tuple(range(H)), tuple(range(H, E_PER))

        h1r_a.wait_recv()
        h2r = [wcopy(bL.at[j], bD.at[j], 4 + j, right) for j in JLO]
        for rd in h2r:
            rd.start()
        out_ref[:, :] = out_ref[:, :] + block_out(bL, left, 1, js=JLO)

        h1l_a.wait_recv()
        h2l = [wcopy(bR.at[j], bD.at[j], 4 + j, left) for j in JHI]
        for rd in h2l:
            rd.start()
        out_ref[:, :] = out_ref[:, :] + block_out(bR, right, 3, js=JHI)

        h1r_b.wait_recv()
        out_ref[:, :] = out_ref[:, :] + block_out(bL, left, 1, js=JHI)
        h1l_b.wait_recv()
        out_ref[:, :] = out_ref[:, :] + block_out(bR, right, 3, js=JLO)

        h2r[0].wait_recv()
        out_ref[:, :] = out_ref[:, :] + block_out(bD, far, 2, js=(0,))
        h2l[0].wait_recv()
        out_ref[:, :] = out_ref[:, :] + block_out(bD, far, 2, js=(2,))
        h2r[1].wait_recv()
        out_ref[:, :] = out_ref[:, :] + block_out(bD, far, 2, js=(1,))
        h2l[1].wait_recv()
        out_ref[:, :] = out_ref[:, :] + block_out(bD, far, 2, js=(3,))

        for rd in (h1r_a, h1l_a, h1r_b, h1l_b, *h2r, *h2l):
            rd.wait_send()

    return pl.pallas_call(
        body,
        out_shape=jax.ShapeDtypeStruct((m_tok, d_ff), jnp.float32),
        in_specs=[
            pl.BlockSpec(memory_space=pltpu.VMEM),
            pl.BlockSpec(memory_space=pltpu.VMEM),
            pl.BlockSpec(memory_space=pltpu.VMEM),
        ],
        out_specs=pl.BlockSpec(memory_space=pltpu.VMEM),
        scratch_shapes=[
            pltpu.VMEM((N_DEV, 1, n_exp + E_PER), jnp.float32),
            pltpu.VMEM((E_PER, d_model, d_ff), jnp.int8),
            pltpu.VMEM((E_PER, d_model, d_ff), jnp.int8),
            pltpu.VMEM((E_PER, d_model, d_ff), jnp.int8),
            pltpu.VMEM((E_PER, d_model, d_ff), jnp.int8),
            pltpu.SemaphoreType.DMA((N_DEV - 1,)),
            pltpu.SemaphoreType.DMA((N_DEV - 1,)),
            pltpu.SemaphoreType.DMA((8,)),
            pltpu.SemaphoreType.DMA((8,)),
        ],
        compiler_params=pltpu.CompilerParams(
            collective_id=0,
            vmem_limit_bytes=64 * 1024 * 1024,
        ),
    )(x, route_idx, expert_W)
